# Initial kernel scaffold; baseline (speedup 1.0000x reference)
import jax
import jax.numpy as jnp
from jax.experimental import pallas as pl


def kernel(user_emb, item_emb, adj_indices, adj_values):
    raise NotImplementedError("write your pallas kernel here")


# speedup vs baseline: 5.1111x; 5.1111x over previous
import jax
import jax.numpy as jnp
from jax import lax
from jax.experimental import pallas as pl
from jax.experimental.pallas import tpu as pltpu
from jax.experimental.pallas import tpu_sc as plsc

N_USERS = 20000
N_ITEMS = 30000
N = N_USERS + N_ITEMS
NP = 50048
HALF = 32
E = 800000
W = 512
G = 128
K = W // G
NT = 16
NW_T = 98
E_PAD = NT * NW_T * W
TROWS = NP // NT
MCH = 128
NMCH = TROWS // MCH
MTAIL = TROWS - NMCH * MCH


def _body(x0_h, rows2_h, cols_h, vals_h, out_h, xs_h, acc, idx, rowv, valv,
          gat):
    cc = lax.axis_index("c")
    ss = lax.axis_index("s")
    IOTA = lax.iota(jnp.int32, 16)
    IOTA16 = IOTA + 16
    ZV = jnp.zeros((16,), jnp.float32)
    cvec = jnp.full((16,), cc * NP, jnp.int32)

    def zero_acc():
        @pl.loop(0, W)
        def _z(i):
            gat[i, pl.ds(0, 16)] = ZV
            gat[i, pl.ds(16, 16)] = ZV
        base = ss * TROWS
        for off in range(0, TROWS - W + 1, W):
            pltpu.sync_copy(gat.at[pl.ds(0, W)],
                            acc.at[pl.ds(base + off, W)])
        rem = TROWS % W
        if rem:
            pltpu.sync_copy(gat.at[pl.ds(0, rem)],
                            acc.at[pl.ds(base + TROWS - rem, rem)])

    def layer(src, dst):
        zero_acc()
        plsc.subcore_barrier()

        @pl.loop(0, NW_T)
        def _w(w):
            e0 = (ss * NW_T + w) * W
            r0 = (ss * NW_T + w) * K
            pltpu.sync_copy(cols_h.at[pl.ds(e0, W)], idx)
            pltpu.sync_copy(rows2_h.at[pl.ds(r0, K)], rowv)
            pltpu.sync_copy(vals_h.at[pl.ds(e0, W)], valv)

            @pl.loop(0, W // 16)
            def _o(i):
                idx[pl.ds(i * 16, 16)] = idx[pl.ds(i * 16, 16)] + cvec

            for g in range(K):
                pltpu.sync_copy(src.at[idx.at[pl.ds(g * G, G)]],
                                gat.at[pl.ds(g * G, G)])

            @pl.loop(0, W // 16)
            def _s(q):
                v16 = valv[pl.ds(q * 16, 16)]
                for j in range(16):
                    sp = jnp.broadcast_to(lax.slice_in_dim(v16, j, j + 1),
                                          (16,))
                    e = q * 16 + j
                    gat[e, pl.ds(0, 16)] = gat[e, pl.ds(0, 16)] * sp
                    gat[e, pl.ds(16, 16)] = gat[e, pl.ds(16, 16)] * sp

            for g in range(K):
                pltpu.sync_copy(gat.at[pl.ds(g * G, G)],
                                acc.at[rowv.at[g]], add=True)

        plsc.subcore_barrier()
        if dst is not None:
            pltpu.sync_copy(acc.at[pl.ds(ss * TROWS, TROWS)],
                            dst.at[pl.ds(cc * NP + ss * TROWS, TROWS)])

    layer(x0_h, xs_h.at[0])
    layer(xs_h.at[0], xs_h.at[1])
    layer(xs_h.at[1], None)

    ro = cc * NP + ss * TROWS

    def mean_chunk(a0, l0, mch):
        pltpu.sync_copy(x0_h.at[pl.ds(a0, mch)], gat.at[pl.ds(0, mch)])
        pltpu.sync_copy(xs_h.at[0, pl.ds(a0, mch)], gat.at[pl.ds(MCH, mch)])
        pltpu.sync_copy(xs_h.at[1, pl.ds(a0, mch)],
                        gat.at[pl.ds(2 * MCH, mch)])
        pltpu.sync_copy(acc.at[pl.ds(l0, mch)], gat.at[pl.ds(3 * MCH, mch)])

        @pl.loop(0, mch)
        def _r(i):
            for h in (0, 16):
                s = (gat[i, pl.ds(h, 16)]
                     + gat[i + MCH, pl.ds(h, 16)]
                     + gat[i + 2 * MCH, pl.ds(h, 16)]
                     + gat[i + 3 * MCH, pl.ds(h, 16)])
                gat[i, pl.ds(h, 16)] = s * 0.25

        pltpu.sync_copy(gat.at[pl.ds(0, mch)], out_h.at[pl.ds(a0, mch)])

    @pl.loop(0, NMCH)
    def _m(j):
        mean_chunk(ro + j * MCH, ss * TROWS + j * MCH, MCH)

    mean_chunk(ro + NMCH * MCH, ss * TROWS + NMCH * MCH, MTAIL)


_gcn = pl.kernel(
    _body,
    out_type=jax.ShapeDtypeStruct((2 * NP, HALF), jnp.float32),
    mesh=plsc.VectorSubcoreMesh(core_axis_name="c", subcore_axis_name="s"),
    compiler_params=pltpu.CompilerParams(use_tc_tiling_on_sc=False),
    scratch_types=[
        pltpu.HBM((2, 2 * NP, HALF), jnp.float32),
        pltpu.VMEM_SHARED((NP, HALF), jnp.float32),
        pltpu.VMEM((W,), jnp.int32),
        pltpu.VMEM((K, G), jnp.int32),
        pltpu.VMEM((W,), jnp.float32),
        pltpu.VMEM((W, HALF), jnp.float32),
    ],
)


def kernel(user_emb, item_emb, adj_indices, adj_values):
    rows = adj_indices[0].astype(jnp.int32)
    cols = adj_indices[1].astype(jnp.int32)
    vals = adj_values.astype(jnp.float32)
    pad = E_PAD - E
    rows = jnp.concatenate([rows, jnp.zeros((pad,), jnp.int32)])
    cols = jnp.concatenate([cols, jnp.zeros((pad,), jnp.int32)])
    vals = jnp.concatenate([vals, jnp.zeros((pad,), jnp.float32)])
    rows2 = rows.reshape(E_PAD // G, G)
    allemb = jnp.concatenate([user_emb, item_emb], axis=0)
    allemb = jnp.concatenate(
        [allemb, jnp.zeros((NP - N, 64), jnp.float32)], axis=0)
    x0 = jnp.concatenate([allemb[:, :HALF], allemb[:, HALF:]], axis=0)
    out = _gcn(x0, rows2, cols, vals)
    users = jnp.concatenate([out[:N_USERS], out[NP: NP + N_USERS]], axis=1)
    items = jnp.concatenate([out[N_USERS:N], out[NP + N_USERS: NP + N]],
                            axis=1)
    return users, items


# speedup vs baseline: 6.8458x; 1.3394x over previous
import jax
import jax.numpy as jnp
from jax import lax
from jax.experimental import pallas as pl
from jax.experimental.pallas import tpu as pltpu
from jax.experimental.pallas import tpu_sc as plsc

N_USERS = 20000
N_ITEMS = 30000
N = N_USERS + N_ITEMS
NP = 50048
HALF = 32
E = 800000
W = 384
G = 128
K = W // G
NT = 16
NW_T = 132
E_PAD = NT * NW_T * W
TROWS = NP // NT
MCH = 96
NMCH = TROWS // MCH
MTAIL = TROWS - NMCH * MCH


def _body(x0_h, rows2_h, cols_h, vals_h, out_h, xs_h, acc,
          idx0, idx1, rowv0, rowv1, valv0, valv1, gat0, gat1,
          esem0, esem1, gsem0, gsem1):
    cc = lax.axis_index("c")
    ss = lax.axis_index("s")
    ZV = jnp.zeros((16,), jnp.float32)
    cvec = jnp.full((16,), cc * NP, jnp.int32)
    idx = (idx0, idx1)
    rowv = (rowv0, rowv1)
    valv = (valv0, valv1)
    gat = (gat0, gat1)
    esem = (esem0, esem1)
    gsem = (gsem0, gsem1)

    def fire_edge(b, wp):
        e0 = (ss * NW_T + wp) * W
        r0 = (ss * NW_T + wp) * K
        pltpu.async_copy(cols_h.at[pl.ds(e0, W)], idx[b], esem[b])
        pltpu.async_copy(rows2_h.at[pl.ds(r0, K)], rowv[b], esem[b])
        pltpu.async_copy(vals_h.at[pl.ds(e0, W)], valv[b], esem[b])

    def wait_edge(b, wp):
        e0 = (ss * NW_T + wp) * W
        r0 = (ss * NW_T + wp) * K
        pltpu.make_async_copy(cols_h.at[pl.ds(e0, W)], idx[b],
                              esem[b]).wait()
        pltpu.make_async_copy(rows2_h.at[pl.ds(r0, K)], rowv[b],
                              esem[b]).wait()
        pltpu.make_async_copy(vals_h.at[pl.ds(e0, W)], valv[b],
                              esem[b]).wait()

    def prep(b, src):
        @pl.loop(0, W // 16)
        def _o(i):
            idx[b][pl.ds(i * 16, 16)] = idx[b][pl.ds(i * 16, 16)] + cvec
        for g in range(K):
            pltpu.async_copy(src.at[idx[b].at[pl.ds(g * G, G)]],
                             gat[b].at[pl.ds(g * G, G)], gsem[b])

    def wait_gath(b, src):
        for g in range(K):
            pltpu.make_async_copy(src.at[idx[b].at[pl.ds(g * G, G)]],
                                  gat[b].at[pl.ds(g * G, G)],
                                  gsem[b]).wait()

    def scale_scatter(b):
        @pl.loop(0, W // 16)
        def _s(q):
            v16 = valv[b][pl.ds(q * 16, 16)]
            for j in range(16):
                sp = jnp.broadcast_to(lax.slice_in_dim(v16, j, j + 1), (16,))
                e = q * 16 + j
                gat[b][e, pl.ds(0, 16)] = gat[b][e, pl.ds(0, 16)] * sp
                gat[b][e, pl.ds(16, 16)] = gat[b][e, pl.ds(16, 16)] * sp
        for g in range(K):
            pltpu.sync_copy(gat[b].at[pl.ds(g * G, G)],
                            acc.at[rowv[b].at[g]], add=True)

    def zero_acc():
        @pl.loop(0, W)
        def _z(i):
            gat0[i, pl.ds(0, 16)] = ZV
            gat0[i, pl.ds(16, 16)] = ZV
        base = ss * TROWS
        for off in range(0, TROWS - W + 1, W):
            pltpu.sync_copy(gat0.at[pl.ds(0, W)],
                            acc.at[pl.ds(base + off, W)])
        rem = TROWS % W
        if rem:
            pltpu.sync_copy(gat0.at[pl.ds(0, rem)],
                            acc.at[pl.ds(base + TROWS - rem, rem)])

    def layer(src, dst):
        zero_acc()
        plsc.subcore_barrier()

        fire_edge(0, 0)
        fire_edge(1, 1)
        wait_edge(0, 0)
        prep(0, src)

        def window_body(wp, b, last):
            wait_gath(b, src)
            scale_scatter(b)
            if not last:
                nb = 1 - b
                wait_edge(nb, wp + 1)
                prep(nb, src)
                fire_edge(b, jnp.minimum(wp + 2, NW_T - 1))

        @pl.loop(0, (NW_T - 2) // 2)
        def _w(j):
            window_body(2 * j, 0, False)
            window_body(2 * j + 1, 1, False)

        window_body(NW_T - 2, 0, False)
        window_body(NW_T - 1, 1, True)
        wait_edge(0, NW_T - 1)

        plsc.subcore_barrier()
        if dst is not None:
            pltpu.sync_copy(acc.at[pl.ds(ss * TROWS, TROWS)],
                            dst.at[pl.ds(cc * NP + ss * TROWS, TROWS)])

    layer(x0_h, xs_h.at[0])
    layer(xs_h.at[0], xs_h.at[1])
    layer(xs_h.at[1], None)

    ro = cc * NP + ss * TROWS

    def mean_chunk(a0, l0, mch):
        pltpu.sync_copy(x0_h.at[pl.ds(a0, mch)], gat0.at[pl.ds(0, mch)])
        pltpu.sync_copy(xs_h.at[0, pl.ds(a0, mch)],
                        gat0.at[pl.ds(MCH, mch)])
        pltpu.sync_copy(xs_h.at[1, pl.ds(a0, mch)],
                        gat0.at[pl.ds(2 * MCH, mch)])
        pltpu.sync_copy(acc.at[pl.ds(l0, mch)], gat0.at[pl.ds(3 * MCH, mch)])

        @pl.loop(0, mch)
        def _r(i):
            for h in (0, 16):
                s = (gat0[i, pl.ds(h, 16)]
                     + gat0[i + MCH, pl.ds(h, 16)]
                     + gat0[i + 2 * MCH, pl.ds(h, 16)]
                     + gat0[i + 3 * MCH, pl.ds(h, 16)])
                gat0[i, pl.ds(h, 16)] = s * 0.25

        pltpu.sync_copy(gat0.at[pl.ds(0, mch)], out_h.at[pl.ds(a0, mch)])

    @pl.loop(0, NMCH)
    def _m(j):
        mean_chunk(ro + j * MCH, ss * TROWS + j * MCH, MCH)

    mean_chunk(ro + NMCH * MCH, ss * TROWS + NMCH * MCH, MTAIL)


_gcn = pl.kernel(
    _body,
    out_type=jax.ShapeDtypeStruct((2 * NP, HALF), jnp.float32),
    mesh=plsc.VectorSubcoreMesh(core_axis_name="c", subcore_axis_name="s"),
    compiler_params=pltpu.CompilerParams(use_tc_tiling_on_sc=False),
    scratch_types=[
        pltpu.HBM((2, 2 * NP, HALF), jnp.float32),
        pltpu.VMEM_SHARED((NP, HALF), jnp.float32),
        pltpu.VMEM((W,), jnp.int32),
        pltpu.VMEM((W,), jnp.int32),
        pltpu.VMEM((K, G), jnp.int32),
        pltpu.VMEM((K, G), jnp.int32),
        pltpu.VMEM((W,), jnp.float32),
        pltpu.VMEM((W,), jnp.float32),
        pltpu.VMEM((W, HALF), jnp.float32),
        pltpu.VMEM((W, HALF), jnp.float32),
        pltpu.SemaphoreType.DMA,
        pltpu.SemaphoreType.DMA,
        pltpu.SemaphoreType.DMA,
        pltpu.SemaphoreType.DMA,
    ],
)


def kernel(user_emb, item_emb, adj_indices, adj_values):
    rows = adj_indices[0].astype(jnp.int32)
    cols = adj_indices[1].astype(jnp.int32)
    vals = adj_values.astype(jnp.float32)
    pad = E_PAD - E
    rows = jnp.concatenate([rows, jnp.zeros((pad,), jnp.int32)])
    cols = jnp.concatenate([cols, jnp.zeros((pad,), jnp.int32)])
    vals = jnp.concatenate([vals, jnp.zeros((pad,), jnp.float32)])
    rows2 = rows.reshape(E_PAD // G, G)
    allemb = jnp.concatenate([user_emb, item_emb], axis=0)
    allemb = jnp.concatenate(
        [allemb, jnp.zeros((NP - N, 64), jnp.float32)], axis=0)
    x0 = jnp.concatenate([allemb[:, :HALF], allemb[:, HALF:]], axis=0)
    out = _gcn(x0, rows2, cols, vals)
    users = jnp.concatenate([out[:N_USERS], out[NP: NP + N_USERS]], axis=1)
    items = jnp.concatenate([out[N_USERS:N], out[NP + N_USERS: NP + N]],
                            axis=1)
    return users, items


# speedup vs baseline: 8.1503x; 1.1905x over previous
import jax
import jax.numpy as jnp
from jax import lax
from jax.experimental import pallas as pl
from jax.experimental.pallas import tpu as pltpu
from jax.experimental.pallas import tpu_sc as plsc

N_USERS = 20000
N_ITEMS = 30000
N = N_USERS + N_ITEMS
NP = 50048
HALF = 32
E = 800000
W = 384
G = 128
K = W // G
NT = 16
NW_T = 132
E_PAD = NT * NW_T * W
TROWS = NP // NT
MCH = 96
NMCH = TROWS // MCH
MTAIL = TROWS - NMCH * MCH


def _body(x0_h, rows2_h, cols_h, vals_h, out_h, xs_h, acc,
          idx0, idx1, rowv0, rowv1, valv0, valv1, gat0, gat1,
          esem0, esem1, gsem0, gsem1, ssem0, ssem1, msem):
    cc = lax.axis_index("c")
    ss = lax.axis_index("s")
    ZV = jnp.zeros((16,), jnp.float32)
    cvec = jnp.full((16,), cc * NP, jnp.int32)
    idx = (idx0, idx1)
    rowv = (rowv0, rowv1)
    valv = (valv0, valv1)
    gat = (gat0, gat1)
    esem = (esem0, esem1)
    gsem = (gsem0, gsem1)
    ssem = (ssem0, ssem1)

    def fire_edge(b, wp):
        e0 = (ss * NW_T + wp) * W
        r0 = (ss * NW_T + wp) * K
        pltpu.async_copy(cols_h.at[pl.ds(e0, W)], idx[b], esem[b])
        pltpu.async_copy(rows2_h.at[pl.ds(r0, K)], rowv[b], esem[b])
        pltpu.async_copy(vals_h.at[pl.ds(e0, W)], valv[b], esem[b])

    def wait_edge(b, wp):
        e0 = (ss * NW_T + wp) * W
        r0 = (ss * NW_T + wp) * K
        pltpu.make_async_copy(cols_h.at[pl.ds(e0, W)], idx[b],
                              esem[b]).wait()
        pltpu.make_async_copy(rows2_h.at[pl.ds(r0, K)], rowv[b],
                              esem[b]).wait()
        pltpu.make_async_copy(vals_h.at[pl.ds(e0, W)], valv[b],
                              esem[b]).wait()

    def prep(b, src):
        @pl.loop(0, W // 16)
        def _o(i):
            idx[b][pl.ds(i * 16, 16)] = idx[b][pl.ds(i * 16, 16)] + cvec
        for g in range(K):
            pltpu.async_copy(src.at[idx[b].at[pl.ds(g * G, G)]],
                             gat[b].at[pl.ds(g * G, G)], gsem[b])

    def wait_gath(b, src):
        for g in range(K):
            pltpu.make_async_copy(src.at[idx[b].at[pl.ds(g * G, G)]],
                                  gat[b].at[pl.ds(g * G, G)],
                                  gsem[b]).wait()

    def scale(b):
        @pl.loop(0, W // 16)
        def _s(q):
            v16 = valv[b][pl.ds(q * 16, 16)]
            for j in range(16):
                sp = jnp.broadcast_to(lax.slice_in_dim(v16, j, j + 1), (16,))
                e = q * 16 + j
                gat[b][e, pl.ds(0, 16)] = gat[b][e, pl.ds(0, 16)] * sp
                gat[b][e, pl.ds(16, 16)] = gat[b][e, pl.ds(16, 16)] * sp

    def fire_scat(b):
        return [pltpu.async_copy(gat[b].at[pl.ds(g * G, G)],
                                 acc.at[rowv[b].at[g]], ssem[b], add=True)
                for g in range(K)]

    def zero_acc():
        @pl.loop(0, W)
        def _z(i):
            gat0[i, pl.ds(0, 16)] = ZV
            gat0[i, pl.ds(16, 16)] = ZV
        base = ss * TROWS
        for off in range(0, TROWS - W + 1, W):
            pltpu.sync_copy(gat0.at[pl.ds(0, W)],
                            acc.at[pl.ds(base + off, W)])
        rem = TROWS % W
        if rem:
            pltpu.sync_copy(gat0.at[pl.ds(0, rem)],
                            acc.at[pl.ds(base + TROWS - rem, rem)])

    def layer(src, dst):
        zero_acc()
        plsc.subcore_barrier()

        fire_edge(0, 0)
        fire_edge(1, 1)
        wait_edge(0, 0)
        prep(0, src)

        def window_body(wp, b, last):
            wait_gath(b, src)
            scale(b)
            sd = fire_scat(b)
            if not last:
                nb = 1 - b
                wait_edge(nb, wp + 1)
                prep(nb, src)
                for d in sd:
                    d.wait()
                fire_edge(b, jnp.minimum(wp + 2, NW_T - 1))
            else:
                for d in sd:
                    d.wait()

        @pl.loop(0, (NW_T - 2) // 2)
        def _w(j):
            window_body(2 * j, 0, False)
            window_body(2 * j + 1, 1, False)

        window_body(NW_T - 2, 0, False)
        window_body(NW_T - 1, 1, True)
        wait_edge(0, NW_T - 1)

        plsc.subcore_barrier()
        if dst is not None:
            pltpu.sync_copy(acc.at[pl.ds(ss * TROWS, TROWS)],
                            dst.at[pl.ds(cc * NP + ss * TROWS, TROWS)])

    layer(x0_h, xs_h.at[0])
    layer(xs_h.at[0], xs_h.at[1])
    layer(xs_h.at[1], None)

    ro = cc * NP + ss * TROWS

    def mean_chunk(a0, l0, mch):
        md = [pltpu.async_copy(x0_h.at[pl.ds(a0, mch)],
                               gat0.at[pl.ds(0, mch)], msem),
              pltpu.async_copy(xs_h.at[0, pl.ds(a0, mch)],
                               gat0.at[pl.ds(MCH, mch)], msem),
              pltpu.async_copy(xs_h.at[1, pl.ds(a0, mch)],
                               gat0.at[pl.ds(2 * MCH, mch)], msem)]
        pltpu.sync_copy(acc.at[pl.ds(l0, mch)], gat0.at[pl.ds(3 * MCH, mch)])
        for d in md:
            d.wait()

        @pl.loop(0, mch)
        def _r(i):
            for h in (0, 16):
                s = (gat0[i, pl.ds(h, 16)]
                     + gat0[i + MCH, pl.ds(h, 16)]
                     + gat0[i + 2 * MCH, pl.ds(h, 16)]
                     + gat0[i + 3 * MCH, pl.ds(h, 16)])
                gat0[i, pl.ds(h, 16)] = s * 0.25

        pltpu.sync_copy(gat0.at[pl.ds(0, mch)], out_h.at[pl.ds(a0, mch)])

    @pl.loop(0, NMCH)
    def _m(j):
        mean_chunk(ro + j * MCH, ss * TROWS + j * MCH, MCH)

    mean_chunk(ro + NMCH * MCH, ss * TROWS + NMCH * MCH, MTAIL)


_gcn = pl.kernel(
    _body,
    out_type=jax.ShapeDtypeStruct((2 * NP, HALF), jnp.float32),
    mesh=plsc.VectorSubcoreMesh(core_axis_name="c", subcore_axis_name="s"),
    compiler_params=pltpu.CompilerParams(use_tc_tiling_on_sc=False),
    scratch_types=[
        pltpu.HBM((2, 2 * NP, HALF), jnp.float32),
        pltpu.VMEM_SHARED((NP, HALF), jnp.float32),
        pltpu.VMEM((W,), jnp.int32),
        pltpu.VMEM((W,), jnp.int32),
        pltpu.VMEM((K, G), jnp.int32),
        pltpu.VMEM((K, G), jnp.int32),
        pltpu.VMEM((W,), jnp.float32),
        pltpu.VMEM((W,), jnp.float32),
        pltpu.VMEM((W, HALF), jnp.float32),
        pltpu.VMEM((W, HALF), jnp.float32),
        pltpu.SemaphoreType.DMA,
        pltpu.SemaphoreType.DMA,
        pltpu.SemaphoreType.DMA,
        pltpu.SemaphoreType.DMA,
        pltpu.SemaphoreType.DMA,
        pltpu.SemaphoreType.DMA,
        pltpu.SemaphoreType.DMA,
    ],
)


def kernel(user_emb, item_emb, adj_indices, adj_values):
    rows = adj_indices[0].astype(jnp.int32)
    cols = adj_indices[1].astype(jnp.int32)
    vals = adj_values.astype(jnp.float32)
    pad = E_PAD - E
    rows = jnp.concatenate([rows, jnp.zeros((pad,), jnp.int32)])
    cols = jnp.concatenate([cols, jnp.zeros((pad,), jnp.int32)])
    vals = jnp.concatenate([vals, jnp.zeros((pad,), jnp.float32)])
    rows2 = rows.reshape(E_PAD // G, G)
    allemb = jnp.concatenate([user_emb, item_emb], axis=0)
    allemb = jnp.concatenate(
        [allemb, jnp.zeros((NP - N, 64), jnp.float32)], axis=0)
    x0 = jnp.concatenate([allemb[:, :HALF], allemb[:, HALF:]], axis=0)
    out = _gcn(x0, rows2, cols, vals)
    users = jnp.concatenate([out[:N_USERS], out[NP: NP + N_USERS]], axis=1)
    items = jnp.concatenate([out[N_USERS:N], out[NP + N_USERS: NP + N]],
                            axis=1)
    return users, items


# speedup vs baseline: 8.1518x; 1.0002x over previous
import jax
import jax.numpy as jnp
from jax import lax
from jax.experimental import pallas as pl
from jax.experimental.pallas import tpu as pltpu
from jax.experimental.pallas import tpu_sc as plsc

N_USERS = 20000
N_ITEMS = 30000
N = N_USERS + N_ITEMS
NP = 50048
HALF = 32
E = 800000
W = 384
G = 128
K = W // G
NT = 16
NW_T = 132
E_PAD = NT * NW_T * W
TROWS = NP // NT
MCH = 96
NMCH = TROWS // MCH
MTAIL = TROWS - NMCH * MCH


def _body(x0_h, rows2_h, cols_h, vals_h, out_h, xs_h, acc,
          idx0, idx1, rowv0, rowv1, valv0, valv1, gat0, gat1,
          esem0, esem1, gsem0, gsem1, ssem0, ssem1, msem):
    cc = lax.axis_index("c")
    ss = lax.axis_index("s")
    ZV = jnp.zeros((16,), jnp.float32)
    cvec = jnp.full((16,), cc * NP, jnp.int32)
    idx = (idx0, idx1)
    rowv = (rowv0, rowv1)
    valv = (valv0, valv1)
    gat = (gat0, gat1)
    esem = (esem0, esem1)
    gsem = (gsem0, gsem1)
    ssem = (ssem0, ssem1)

    def fire_edge(b, wp):
        e0 = (ss * NW_T + wp) * W
        r0 = ss * NW_T + wp
        pltpu.async_copy(cols_h.at[pl.ds(e0, W)], idx[b], esem[b])
        pltpu.async_copy(rows2_h.at[pl.ds(r0, 1)], rowv[b], esem[b])
        pltpu.async_copy(vals_h.at[pl.ds(e0, W)], valv[b], esem[b])

    def wait_edge(b, wp):
        e0 = (ss * NW_T + wp) * W
        r0 = ss * NW_T + wp
        pltpu.make_async_copy(cols_h.at[pl.ds(e0, W)], idx[b],
                              esem[b]).wait()
        pltpu.make_async_copy(rows2_h.at[pl.ds(r0, 1)], rowv[b],
                              esem[b]).wait()
        pltpu.make_async_copy(vals_h.at[pl.ds(e0, W)], valv[b],
                              esem[b]).wait()

    def prep(b, src):
        @pl.loop(0, W // 16)
        def _o(i):
            idx[b][pl.ds(i * 16, 16)] = idx[b][pl.ds(i * 16, 16)] + cvec
        pltpu.async_copy(src.at[idx[b]], gat[b], gsem[b])

    def wait_gath(b, src):
        pltpu.make_async_copy(src.at[idx[b]], gat[b], gsem[b]).wait()

    def scale(b):
        @pl.loop(0, W // 16)
        def _s(q):
            v16 = valv[b][pl.ds(q * 16, 16)]
            for j in range(16):
                sp = jnp.broadcast_to(lax.slice_in_dim(v16, j, j + 1), (16,))
                e = q * 16 + j
                gat[b][e, pl.ds(0, 16)] = gat[b][e, pl.ds(0, 16)] * sp
                gat[b][e, pl.ds(16, 16)] = gat[b][e, pl.ds(16, 16)] * sp

    def fire_scat(b):
        return [pltpu.async_copy(gat[b], acc.at[rowv[b].at[0]], ssem[b],
                                 add=True)]

    def zero_acc():
        @pl.loop(0, W)
        def _z(i):
            gat0[i, pl.ds(0, 16)] = ZV
            gat0[i, pl.ds(16, 16)] = ZV
        base = ss * TROWS
        for off in range(0, TROWS - W + 1, W):
            pltpu.sync_copy(gat0.at[pl.ds(0, W)],
                            acc.at[pl.ds(base + off, W)])
        rem = TROWS % W
        if rem:
            pltpu.sync_copy(gat0.at[pl.ds(0, rem)],
                            acc.at[pl.ds(base + TROWS - rem, rem)])

    def layer(src, dst):
        zero_acc()
        plsc.subcore_barrier()

        fire_edge(0, 0)
        fire_edge(1, 1)
        wait_edge(0, 0)
        prep(0, src)

        def window_body(wp, b, last):
            wait_gath(b, src)
            scale(b)
            sd = fire_scat(b)
            if not last:
                nb = 1 - b
                wait_edge(nb, wp + 1)
                prep(nb, src)
                for d in sd:
                    d.wait()
                fire_edge(b, jnp.minimum(wp + 2, NW_T - 1))
            else:
                for d in sd:
                    d.wait()

        @pl.loop(0, (NW_T - 2) // 2)
        def _w(j):
            window_body(2 * j, 0, False)
            window_body(2 * j + 1, 1, False)

        window_body(NW_T - 2, 0, False)
        window_body(NW_T - 1, 1, True)
        wait_edge(0, NW_T - 1)

        plsc.subcore_barrier()
        if dst is not None:
            pltpu.sync_copy(acc.at[pl.ds(ss * TROWS, TROWS)],
                            dst.at[pl.ds(cc * NP + ss * TROWS, TROWS)])

    layer(x0_h, xs_h.at[0])
    layer(xs_h.at[0], xs_h.at[1])
    layer(xs_h.at[1], None)

    ro = cc * NP + ss * TROWS

    def mean_chunk(a0, l0, mch):
        md = [pltpu.async_copy(x0_h.at[pl.ds(a0, mch)],
                               gat0.at[pl.ds(0, mch)], msem),
              pltpu.async_copy(xs_h.at[0, pl.ds(a0, mch)],
                               gat0.at[pl.ds(MCH, mch)], msem),
              pltpu.async_copy(xs_h.at[1, pl.ds(a0, mch)],
                               gat0.at[pl.ds(2 * MCH, mch)], msem)]
        pltpu.sync_copy(acc.at[pl.ds(l0, mch)], gat0.at[pl.ds(3 * MCH, mch)])
        for d in md:
            d.wait()

        @pl.loop(0, mch)
        def _r(i):
            for h in (0, 16):
                s = (gat0[i, pl.ds(h, 16)]
                     + gat0[i + MCH, pl.ds(h, 16)]
                     + gat0[i + 2 * MCH, pl.ds(h, 16)]
                     + gat0[i + 3 * MCH, pl.ds(h, 16)])
                gat0[i, pl.ds(h, 16)] = s * 0.25

        pltpu.sync_copy(gat0.at[pl.ds(0, mch)], out_h.at[pl.ds(a0, mch)])

    @pl.loop(0, NMCH)
    def _m(j):
        mean_chunk(ro + j * MCH, ss * TROWS + j * MCH, MCH)

    mean_chunk(ro + NMCH * MCH, ss * TROWS + NMCH * MCH, MTAIL)


_gcn = pl.kernel(
    _body,
    out_type=jax.ShapeDtypeStruct((2 * NP, HALF), jnp.float32),
    mesh=plsc.VectorSubcoreMesh(core_axis_name="c", subcore_axis_name="s"),
    compiler_params=pltpu.CompilerParams(use_tc_tiling_on_sc=False),
    scratch_types=[
        pltpu.HBM((2, 2 * NP, HALF), jnp.float32),
        pltpu.VMEM_SHARED((NP, HALF), jnp.float32),
        pltpu.VMEM((W,), jnp.int32),
        pltpu.VMEM((W,), jnp.int32),
        pltpu.VMEM((1, W), jnp.int32),
        pltpu.VMEM((1, W), jnp.int32),
        pltpu.VMEM((W,), jnp.float32),
        pltpu.VMEM((W,), jnp.float32),
        pltpu.VMEM((W, HALF), jnp.float32),
        pltpu.VMEM((W, HALF), jnp.float32),
        pltpu.SemaphoreType.DMA,
        pltpu.SemaphoreType.DMA,
        pltpu.SemaphoreType.DMA,
        pltpu.SemaphoreType.DMA,
        pltpu.SemaphoreType.DMA,
        pltpu.SemaphoreType.DMA,
        pltpu.SemaphoreType.DMA,
    ],
)


def kernel(user_emb, item_emb, adj_indices, adj_values):
    rows = adj_indices[0].astype(jnp.int32)
    cols = adj_indices[1].astype(jnp.int32)
    vals = adj_values.astype(jnp.float32)
    pad = E_PAD - E
    rows = jnp.concatenate([rows, jnp.zeros((pad,), jnp.int32)])
    cols = jnp.concatenate([cols, jnp.zeros((pad,), jnp.int32)])
    vals = jnp.concatenate([vals, jnp.zeros((pad,), jnp.float32)])
    rows2 = rows.reshape(E_PAD // W, W)
    allemb = jnp.concatenate([user_emb, item_emb], axis=0)
    allemb = jnp.concatenate(
        [allemb, jnp.zeros((NP - N, 64), jnp.float32)], axis=0)
    x0 = jnp.concatenate([allemb[:, :HALF], allemb[:, HALF:]], axis=0)
    out = _gcn(x0, rows2, cols, vals)
    users = jnp.concatenate([out[:N_USERS], out[NP: NP + N_USERS]], axis=1)
    items = jnp.concatenate([out[N_USERS:N], out[NP + N_USERS: NP + N]],
                            axis=1)
    return users, items


# speedup vs baseline: 9.5793x; 1.1751x over previous
import jax
import jax.numpy as jnp
from jax import lax
from jax.experimental import pallas as pl
from jax.experimental.pallas import tpu as pltpu
from jax.experimental.pallas import tpu_sc as plsc

N_USERS = 20000
N_ITEMS = 30000
N = N_USERS + N_ITEMS
NP = 50048
HALF = 16
E = 800000
W = 768
NT = 16
NW_T = 66
E_PAD = NT * NW_T * W
TROWS = NP // NT
MCH = 96
NMCH = TROWS // MCH
MTAIL = TROWS - NMCH * MCH


def _body(x0_h, epack_h, out_h, xs_h, xsp, eb0, eb1, gat0, gat1,
          esem0, esem1, gsem0, gsem1, ssem0, ssem1, msem):
    cc = lax.axis_index("c")
    ss = lax.axis_index("s")
    ZV = jnp.zeros((16,), jnp.float32)
    eb = (eb0, eb1)
    gat = (gat0, gat1)
    esem = (esem0, esem1)
    gsem = (gsem0, gsem1)
    ssem = (ssem0, ssem1)

    def fire_edge(b, wp):
        r0 = ss * NW_T + wp
        pltpu.async_copy(epack_h.at[pl.ds(r0, 1)], eb[b], esem[b])

    def wait_edge(b, wp):
        r0 = ss * NW_T + wp
        pltpu.make_async_copy(epack_h.at[pl.ds(r0, 1)], eb[b],
                              esem[b]).wait()

    def prep(b, li):
        pltpu.async_copy(xsp.at[li].at[eb[b].at[0, 0]], gat[b], gsem[b])

    def wait_gath(b, li):
        pltpu.make_async_copy(xsp.at[li].at[eb[b].at[0, 0]], gat[b],
                              gsem[b]).wait()

    def scale(b):
        @pl.loop(0, W // 16)
        def _s(q):
            v16 = lax.bitcast_convert_type(
                eb[b][0, 2, pl.ds(q * 16, 16)], jnp.float32)
            for j in range(16):
                sp = jnp.broadcast_to(lax.slice_in_dim(v16, j, j + 1), (16,))
                e = q * 16 + j
                gat[b][e, pl.ds(0, 16)] = gat[b][e, pl.ds(0, 16)] * sp

    def fire_scat(b, lo):
        return [pltpu.async_copy(gat[b], xsp.at[lo].at[eb[b].at[0, 1]],
                                 ssem[b], add=True)]

    def zero_buf(lo):
        @pl.loop(0, W)
        def _z(i):
            gat0[i, pl.ds(0, 16)] = ZV
        base = ss * TROWS
        for off in range(0, TROWS - W + 1, W):
            pltpu.sync_copy(gat0.at[pl.ds(0, W)],
                            xsp.at[lo].at[pl.ds(base + off, W)])
        rem = TROWS % W
        if rem:
            pltpu.sync_copy(gat0.at[pl.ds(0, rem)],
                            xsp.at[lo].at[pl.ds(base + TROWS - rem, rem)])

    def layer(li, lo, dst):
        zero_buf(lo)
        plsc.subcore_barrier()

        fire_edge(0, 0)
        fire_edge(1, 1)
        wait_edge(0, 0)
        prep(0, li)

        def window_body(wp, b, last):
            wait_gath(b, li)
            scale(b)
            sd = fire_scat(b, lo)
            if not last:
                nb = 1 - b
                wait_edge(nb, wp + 1)
                prep(nb, li)
                for d in sd:
                    d.wait()
                fire_edge(b, jnp.minimum(wp + 2, NW_T - 1))
            else:
                for d in sd:
                    d.wait()

        @pl.loop(0, (NW_T - 2) // 2)
        def _w(j):
            window_body(2 * j, 0, False)
            window_body(2 * j + 1, 1, False)

        window_body(NW_T - 2, 0, False)
        window_body(NW_T - 1, 1, True)
        wait_edge(0, NW_T - 1)

        plsc.subcore_barrier()
        if dst is not None:
            pltpu.sync_copy(xsp.at[lo].at[pl.ds(ss * TROWS, TROWS)],
                            dst.at[pl.ds(ss * TROWS, TROWS)])

    def mean_chunk(qb, a0, x3b, mch):
        md = [pltpu.async_copy(x0_h.at[pl.ds(qb + a0, mch)],
                               gat0.at[pl.ds(0, mch)], msem),
              pltpu.async_copy(xs_h.at[0, pl.ds(qb + a0, mch)],
                               gat0.at[pl.ds(MCH, mch)], msem),
              pltpu.async_copy(xs_h.at[1, pl.ds(qb + a0, mch)],
                               gat0.at[pl.ds(2 * MCH, mch)], msem)]
        pltpu.sync_copy(xsp.at[x3b].at[pl.ds(a0, mch)],
                        gat0.at[pl.ds(3 * MCH, mch)])
        for d in md:
            d.wait()

        @pl.loop(0, mch)
        def _r(i):
            s = (gat0[i, pl.ds(0, 16)]
                 + gat0[i + MCH, pl.ds(0, 16)]
                 + gat0[i + 2 * MCH, pl.ds(0, 16)]
                 + gat0[i + 3 * MCH, pl.ds(0, 16)])
            gat0[i, pl.ds(0, 16)] = s * 0.25

        pltpu.sync_copy(gat0.at[pl.ds(0, mch)],
                        out_h.at[pl.ds(qb + a0, mch)])

    for q in (0, 1):
        cq = cc * 2 + q
        qb = cq * NP
        pltpu.sync_copy(x0_h.at[pl.ds(qb + ss * TROWS, TROWS)],
                        xsp.at[0].at[pl.ds(ss * TROWS, TROWS)])
        plsc.subcore_barrier()
        layer(0, 1, xs_h.at[0].at[pl.ds(qb, NP)])
        layer(1, 0, xs_h.at[1].at[pl.ds(qb, NP)])
        layer(0, 1, None)

        ts = ss * TROWS

        @pl.loop(0, NMCH)
        def _m(j):
            mean_chunk(qb, ts + j * MCH, 1, MCH)

        mean_chunk(qb, ts + NMCH * MCH, 1, MTAIL)
        plsc.subcore_barrier()


_gcn = pl.kernel(
    _body,
    out_type=jax.ShapeDtypeStruct((4 * NP, HALF), jnp.float32),
    mesh=plsc.VectorSubcoreMesh(core_axis_name="c", subcore_axis_name="s"),
    compiler_params=pltpu.CompilerParams(use_tc_tiling_on_sc=False),
    scratch_types=[
        pltpu.HBM((2, 4 * NP, HALF), jnp.float32),
        pltpu.VMEM_SHARED((2, NP, HALF), jnp.float32),
        pltpu.VMEM((1, 3, W), jnp.int32),
        pltpu.VMEM((1, 3, W), jnp.int32),
        pltpu.VMEM((W, HALF), jnp.float32),
        pltpu.VMEM((W, HALF), jnp.float32),
        pltpu.SemaphoreType.DMA,
        pltpu.SemaphoreType.DMA,
        pltpu.SemaphoreType.DMA,
        pltpu.SemaphoreType.DMA,
        pltpu.SemaphoreType.DMA,
        pltpu.SemaphoreType.DMA,
        pltpu.SemaphoreType.DMA,
    ],
)


def kernel(user_emb, item_emb, adj_indices, adj_values):
    rows = adj_indices[0].astype(jnp.int32)
    cols = adj_indices[1].astype(jnp.int32)
    vals = adj_values.astype(jnp.float32)
    pad = E_PAD - E
    rows = jnp.concatenate([rows, jnp.zeros((pad,), jnp.int32)])
    cols = jnp.concatenate([cols, jnp.zeros((pad,), jnp.int32)])
    vals = jnp.concatenate([vals, jnp.zeros((pad,), jnp.float32)])
    vals_i = lax.bitcast_convert_type(vals, jnp.int32)
    epack = jnp.stack([cols, rows, vals_i]).reshape(3, E_PAD // W, W)
    epack = jnp.transpose(epack, (1, 0, 2))
    allemb = jnp.concatenate([user_emb, item_emb], axis=0)
    allemb = jnp.concatenate(
        [allemb, jnp.zeros((NP - N, 64), jnp.float32)], axis=0)
    x0 = jnp.transpose(allemb.reshape(NP, 4, HALF), (1, 0, 2))
    x0 = x0.reshape(4 * NP, HALF)
    out = _gcn(x0, epack)
    full = jnp.transpose(out.reshape(4, NP, HALF), (1, 0, 2)).reshape(NP, 64)
    return full[:N_USERS], full[N_USERS:N]
